# baseline (speedup 1.0000x reference)
import jax
import jax.numpy as jnp
from jax import lax
from jax.experimental import pallas as pl
from jax.experimental.pallas import tpu as pltpu
from jax.experimental.pallas import tpu_sc as plsc

_VOCAB = 1000000
_FIELDS = 26
_BATCH = 16384

_info = plsc.get_sparse_core_info()
_NC, _NS, _L = _info.num_cores, _info.num_subcores, _info.num_lanes
_NW = _NC * _NS
_BPW = _BATCH // _NW
_IPW = _BPW * _FIELDS


def _body(table_hbm, idx_hbm, out_hbm, idx_v, vals_v, out_v, sem, gsem):
    wid = lax.axis_index("s") * _NC + lax.axis_index("c")
    row0 = wid * _BPW

    cps = [
        pltpu.async_copy(
            idx_hbm.at[j].at[pl.ds(row0, _BPW)],
            idx_v.at[pl.ds(j * _BPW, _BPW)],
            sem,
        )
        for j in range(_FIELDS)
    ]
    gps = []
    for j in range(_FIELDS):
        cps[j].wait()
        gps.append(
            pltpu.async_copy(
                table_hbm.at[0].at[idx_v.at[pl.ds(j * _BPW, _BPW)]],
                vals_v.at[pl.ds(j * _BPW, _BPW)],
                gsem,
            )
        )

    def zinit(c, carry):
        out_v[pl.ds(c * _L, _L)] = jnp.zeros((_L,), jnp.float32)
        return carry

    lax.fori_loop(0, _BPW // _L, zinit, 0)

    for j in range(_FIELDS):
        gps[j].wait()

        def facc(c, carry, j=j):
            o = pl.ds(c * _L, _L)
            out_v[o] = out_v[o] + vals_v[pl.ds(j * _BPW + c * _L, _L)]
            return carry

        lax.fori_loop(0, _BPW // _L, facc, 0)

    def fin(c, carry):
        o = pl.ds(c * _L, _L)
        m = out_v[o] * (1.0 / _FIELDS)
        out_v[o] = 1.0 / (1.0 + jnp.exp(-m))
        return carry

    lax.fori_loop(0, _BPW // _L, fin, 0)

    pltpu.sync_copy(out_v, out_hbm.at[pl.ds(row0, _BPW)])


def kernel(inputs, table):
    idx = lax.optimization_barrier(inputs.T)
    tab = lax.optimization_barrier(table.T)
    mesh = plsc.VectorSubcoreMesh(core_axis_name="c", subcore_axis_name="s")
    run = pl.kernel(
        _body,
        out_type=jax.ShapeDtypeStruct((_BATCH,), jnp.float32),
        mesh=mesh,
        scratch_types=[
            pltpu.VMEM((_IPW,), jnp.int32),
            pltpu.VMEM((_IPW,), jnp.float32),
            pltpu.VMEM((_BPW,), jnp.float32),
            pltpu.SemaphoreType.DMA,
            pltpu.SemaphoreType.DMA,
        ],
    )
    out = run(tab, idx)
    return out.reshape(_BATCH, 1)


# speedup vs baseline: 1.0941x; 1.0941x over previous
import jax
import jax.numpy as jnp
from jax import lax
from jax.experimental import pallas as pl
from jax.experimental.pallas import tpu as pltpu
from jax.experimental.pallas import tpu_sc as plsc

_VOCAB = 1000000
_FIELDS = 26
_BATCH = 16384

_info = plsc.get_sparse_core_info()
_NC, _NS, _L = _info.num_cores, _info.num_subcores, _info.num_lanes
_NW = _NC * _NS
_BPW = _BATCH // _NW
_IPW = _BPW * _FIELDS


def _body(table_hbm, idx_hbm, out_hbm, idx_v, vals_v, out_v, sem, gsem):
    wid = lax.axis_index("s") * _NC + lax.axis_index("c")
    row0 = wid * _BPW

    cps = [
        pltpu.async_copy(
            idx_hbm.at[j].at[pl.ds(row0, _BPW)],
            idx_v.at[pl.ds(j * _BPW, _BPW)],
            sem,
        )
        for j in range(_FIELDS)
    ]
    gps = []
    for j in range(_FIELDS):
        cps[j].wait()
        gps.append(
            pltpu.async_copy(
                table_hbm.at[0].at[idx_v.at[pl.ds(j * _BPW, _BPW)]],
                vals_v.at[pl.ds(j * _BPW, _BPW)],
                gsem,
            )
        )

    for gp in gps:
        gp.wait()

    def chunk(c, carry):
        acc = jnp.zeros((_L,), jnp.float32)
        for j in range(_FIELDS):
            acc = acc + vals_v[pl.ds(j * _BPW + c * _L, _L)]
        m = acc * (1.0 / _FIELDS)
        out_v[pl.ds(c * _L, _L)] = 1.0 / (1.0 + jnp.exp(-m))
        return carry

    lax.fori_loop(0, _BPW // _L, chunk, 0)

    pltpu.sync_copy(out_v, out_hbm.at[pl.ds(row0, _BPW)])


def kernel(inputs, table):
    idx = lax.optimization_barrier(inputs.T)
    tab = lax.optimization_barrier(table.T)
    mesh = plsc.VectorSubcoreMesh(core_axis_name="c", subcore_axis_name="s")
    run = pl.kernel(
        _body,
        out_type=jax.ShapeDtypeStruct((_BATCH,), jnp.float32),
        mesh=mesh,
        scratch_types=[
            pltpu.VMEM((_IPW,), jnp.int32),
            pltpu.VMEM((_IPW,), jnp.float32),
            pltpu.VMEM((_BPW,), jnp.float32),
            pltpu.SemaphoreType.DMA,
            pltpu.SemaphoreType.DMA,
        ],
    )
    out = run(tab, idx)
    return out.reshape(_BATCH, 1)
